# baseline (speedup 1.0000x reference)
import functools

import jax
import jax.numpy as jnp
from jax import lax
from jax.experimental import pallas as pl
from jax.experimental.pallas import tpu as pltpu
from jax.experimental.pallas import tpu_sc as plsc

_CHUNK = 104


def _sc_pool(tables, x_pad, B, D, n_chunks):
    info = plsc.get_sparse_core_info()
    NC, NS = info.num_cores, info.num_subcores
    NW = NC * NS
    b_per_w = B // NW
    n_half = D // 16

    mesh = plsc.VectorSubcoreMesh(core_axis_name="c", subcore_axis_name="s")

    @functools.partial(
        pl.kernel,
        out_type=jax.ShapeDtypeStruct((B, 3 * D), jnp.float32),
        mesh=mesh,
        compiler_params=pltpu.CompilerParams(use_tc_tiling_on_sc=False),
        scratch_types=[
            pltpu.VMEM((b_per_w, n_chunks, _CHUNK), jnp.int32),
            pltpu.VMEM((3 * n_chunks, _CHUNK, D), jnp.float32),
            pltpu.VMEM((3 * n_chunks, _CHUNK, D), jnp.float32),
            pltpu.VMEM((b_per_w, 3 * D), jnp.float32),
            pltpu.SemaphoreType.DMA,
            pltpu.SemaphoreType.DMA,
        ],
    )
    def pool(t0_hbm, t1_hbm, t2_hbm, x_hbm, out_hbm,
             idx_v, buf_a, buf_b, out_v, sem_a, sem_b):
        tbls = (t0_hbm, t1_hbm, t2_hbm)
        wid = lax.axis_index("s") * NC + lax.axis_index("c")
        base = wid * b_per_w
        pltpu.sync_copy(x_hbm.at[pl.ds(base, b_per_w)], idx_v)

        def fire(r, buf, sem):
            for t in range(3):
                for c in range(n_chunks):
                    pltpu.async_copy(
                        tbls[t].at[idx_v.at[r, c]], buf.at[t * n_chunks + c], sem)

        def drain(r, buf, sem):
            for t in range(3):
                for c in range(n_chunks):
                    pltpu.make_async_copy(
                        tbls[t].at[idx_v.at[r, c]], buf.at[t * n_chunks + c], sem
                    ).wait()

        def reduce_store(r, buf):
            for t in range(3):
                accs = tuple(jnp.zeros((16,), jnp.float32) for _ in range(n_half))
                for c in range(n_chunks):
                    i = t * n_chunks + c

                    def body(j, a, i=i):
                        return tuple(
                            a[h] + buf[i, j, pl.ds(16 * h, 16)] for h in range(n_half))

                    accs = lax.fori_loop(0, _CHUNK, body, accs)
                for h in range(n_half):
                    out_v[r, pl.ds(t * D + 16 * h, 16)] = accs[h]

        fire(0, buf_a, sem_a)

        def loop_body(k, carry):
            r_a = 2 * k
            r_b = 2 * k + 1
            fire(r_b, buf_b, sem_b)
            drain(r_a, buf_a, sem_a)
            reduce_store(r_a, buf_a)

            @pl.when(r_a + 2 < b_per_w)
            def _():
                fire(r_a + 2, buf_a, sem_a)

            drain(r_b, buf_b, sem_b)
            reduce_store(r_b, buf_b)
            return carry

        lax.fori_loop(0, b_per_w // 2, loop_body, 0)
        pltpu.sync_copy(out_v, out_hbm.at[pl.ds(base, b_per_w)])

    return pool(tables[0], tables[1], tables[2], x_pad)


def kernel(x, w_word, w_bigram, w_trigram, w1, b1, w2, b2):
    B, S = x.shape
    V, D = w_word.shape
    H = w1.shape[1]
    L = w2.shape[1]
    PAD = V - 1

    S_pad = ((S + _CHUNK - 1) // _CHUNK) * _CHUNK
    n_chunks = S_pad // _CHUNK
    x_pad = jnp.pad(x, ((0, 0), (0, S_pad - S)), constant_values=PAD)
    x_pad = x_pad.reshape(B, n_chunks, _CHUNK)

    pooled = _sc_pool((w_word, w_bigram, w_trigram), x_pad, B, D, n_chunks)

    inv_s = 1.0 / S

    def mlp_body(p_ref, w1_ref, b1_ref, w2_ref, b2_ref, o_ref):
        acts = p_ref[...] * inv_s
        h = jnp.dot(acts, w1_ref[...], preferred_element_type=jnp.float32)
        h = h + b1_ref[...]
        o = jnp.dot(h, w2_ref[...], preferred_element_type=jnp.float32)
        o = o + b2_ref[...]
        o_ref[...] = jnp.maximum(o, 0.0)

    return pl.pallas_call(
        mlp_body,
        out_shape=jax.ShapeDtypeStruct((B, L), jnp.float32),
    )(pooled, w1, b1.reshape(1, H), w2, b2.reshape(1, L))


# speedup vs baseline: 4.3988x; 4.3988x over previous
import functools

import jax
import jax.numpy as jnp
from jax import lax
from jax.experimental import pallas as pl
from jax.experimental.pallas import tpu as pltpu
from jax.experimental.pallas import tpu_sc as plsc

_CHUNK = 112
_BN = 8192


def _project_tables(wt0, wt1, wt2, w1, b1_2d, w2, b2_2d, V, D, H, L):
    grid = pl.cdiv(V, _BN)

    def body(t0_ref, t1_ref, t2_ref, w1_ref, b1_ref, w2_ref, b2_ref,
             p0_ref, p1_ref, c0_ref):
        wc = jnp.dot(w1_ref[...], w2_ref[...],
                     preferred_element_type=jnp.float32)
        dn = (((0,), (0,)), ((), ()))
        acc = lax.dot_general(wc[0:D], t0_ref[...], dn,
                              preferred_element_type=jnp.float32)
        acc += lax.dot_general(wc[D:2 * D], t1_ref[...], dn,
                               preferred_element_type=jnp.float32)
        acc += lax.dot_general(wc[2 * D:3 * D], t2_ref[...], dn,
                               preferred_element_type=jnp.float32)
        p0_ref[...] = acc[0]
        p1_ref[...] = acc[1]

        @pl.when(pl.program_id(0) == 0)
        def _():
            c0_ref[...] = (jnp.dot(b1_ref[...], w2_ref[...],
                                   preferred_element_type=jnp.float32)
                           + b2_ref[...])

    return pl.pallas_call(
        body,
        grid=(grid,),
        in_specs=[
            pl.BlockSpec((D, _BN), lambda j: (0, j)),
            pl.BlockSpec((D, _BN), lambda j: (0, j)),
            pl.BlockSpec((D, _BN), lambda j: (0, j)),
            pl.BlockSpec((3 * D, H), lambda j: (0, 0)),
            pl.BlockSpec((1, H), lambda j: (0, 0)),
            pl.BlockSpec((H, L), lambda j: (0, 0)),
            pl.BlockSpec((1, L), lambda j: (0, 0)),
        ],
        out_specs=[
            pl.BlockSpec((_BN,), lambda j: (j,)),
            pl.BlockSpec((_BN,), lambda j: (j,)),
            pl.BlockSpec((1, L), lambda j: (0, 0)),
        ],
        out_shape=[
            jax.ShapeDtypeStruct((V,), jnp.float32),
            jax.ShapeDtypeStruct((V,), jnp.float32),
            jax.ShapeDtypeStruct((1, L), jnp.float32),
        ],
    )(wt0, wt1, wt2, w1, b1_2d, w2, b2_2d)


def _sc_pool(p0, p1, xi, B, n_chunks):
    info = plsc.get_sparse_core_info()
    NC, NS = info.num_cores, info.num_subcores
    b_per_w = B // (NC * NS)
    n_str = 2 * n_chunks

    mesh = plsc.VectorSubcoreMesh(core_axis_name="c", subcore_axis_name="s")

    @functools.partial(
        pl.kernel,
        out_type=jax.ShapeDtypeStruct((B, 32), jnp.float32),
        mesh=mesh,
        scratch_types=[
            pltpu.VMEM((b_per_w, n_chunks, _CHUNK), jnp.int32),
            pltpu.VMEM((b_per_w, n_str, _CHUNK), jnp.float32),
            pltpu.VMEM((b_per_w, 32), jnp.float32),
            pltpu.SemaphoreType.DMA,
        ],
    )
    def pool(p0_hbm, p1_hbm, xi_hbm, out_hbm, idx_v, buf, out_v, sem):
        tbls = (p0_hbm, p1_hbm)
        wid = lax.axis_index("s") * NC + lax.axis_index("c")
        base = wid * b_per_w
        pltpu.sync_copy(xi_hbm.at[pl.ds(base, b_per_w)], idx_v)

        def copies(r):
            for t in range(2):
                for c in range(n_chunks):
                    yield tbls[t].at[idx_v.at[r, c]], buf.at[r, t * n_chunks + c]

        def fire(r, _):
            for src, dst in copies(r):
                pltpu.async_copy(src, dst, sem)
            return _

        def drain(r, _):
            for src, dst in copies(r):
                pltpu.make_async_copy(src, dst, sem).wait()
            return _

        def reduce(r, _):
            for t in range(2):
                acc = jnp.zeros((16,), jnp.float32)
                for c in range(n_chunks):
                    for j in range(_CHUNK // 16):
                        acc = acc + buf[r, t * n_chunks + c, pl.ds(16 * j, 16)]
                out_v[r, pl.ds(16 * t, 16)] = acc
            return _

        lax.fori_loop(0, b_per_w, fire, 0)
        lax.fori_loop(0, b_per_w, drain, 0)
        lax.fori_loop(0, b_per_w, reduce, 0)
        pltpu.sync_copy(out_v, out_hbm.at[pl.ds(base, b_per_w)])

    return pool(p0, p1, xi)


def kernel(x, w_word, w_bigram, w_trigram, w1, b1, w2, b2):
    B, S = x.shape
    V, D = w_word.shape
    H = w1.shape[1]
    L = w2.shape[1]
    PAD = V - 1

    p0, p1, c0 = _project_tables(
        w_word.T, w_bigram.T, w_trigram.T,
        w1, b1.reshape(1, H), w2, b2.reshape(1, L), V, D, H, L)

    S_pad = ((S + _CHUNK - 1) // _CHUNK) * _CHUNK
    n_chunks = S_pad // _CHUNK
    xi = jnp.pad(x, ((0, 0), (0, S_pad - S)), constant_values=PAD)
    xi = xi.reshape(B, n_chunks, _CHUNK)

    pooled = _sc_pool(p0, p1, xi, B, n_chunks)

    inv_s = 1.0 / S

    def fin_body(p_ref, c0_ref, o_ref):
        rows = lax.broadcasted_iota(jnp.int32, (32, L), 0)
        cols = lax.broadcasted_iota(jnp.int32, (32, L), 1)
        sel = jnp.where(rows // 16 == cols, 1.0, 0.0)
        o = jnp.dot(p_ref[...], sel, preferred_element_type=jnp.float32)
        o_ref[...] = jnp.maximum(o * inv_s + c0_ref[...], 0.0)

    return pl.pallas_call(
        fin_body,
        out_shape=jax.ShapeDtypeStruct((B, L), jnp.float32),
    )(pooled, c0)
